# Initial kernel scaffold; baseline (speedup 1.0000x reference)
import jax
import jax.numpy as jnp
from jax.experimental import pallas as pl


def kernel(points, features, neighbor_indices):
    raise NotImplementedError("write your pallas kernel here")


# speedup vs baseline: 1.4247x; 1.4247x over previous
import functools

import jax
import jax.numpy as jnp
from jax import lax
from jax.experimental import pallas as pl
from jax.experimental.pallas import tpu as pltpu
from jax.experimental.pallas import tpu_sc as plsc

N = 50000
F = 128
P = 25000
K = 16

NC = 2
NS = 16
NW = NC * NS

P_PAD = 25600
PPW = P_PAD // NW
CPTS = 8
NCHUNK = PPW // CPTS
ROWS = CPTS * K
LANES = 16


def _pool_body(features_hbm, idx_hbm, out_hbm, idx_v, rows_v, out_v, sem0, sem1):
    wid = lax.axis_index("s") * NC + lax.axis_index("c")
    base = wid * PPW
    pltpu.sync_copy(idx_hbm.at[wid], idx_v)

    sems = (sem0, sem1)

    def start(g, b):
        pltpu.make_async_copy(
            features_hbm.at[idx_v.at[g]], rows_v.at[b], sems[b]
        ).start()

    def wait(b):
        pltpu.make_async_copy(
            features_hbm.at[idx_v.at[0]], rows_v.at[b], sems[b]
        ).wait()

    start(0, 0)
    start(1, 1)

    @pl.loop(0, NCHUNK, step=2)
    def _chunks(g2):
        for b in range(2):
            g = g2 + b
            wait(b)
            for i in range(CPTS):
                r0 = i * K
                for j in range(F // LANES):
                    vals = [
                        rows_v[b, r0 + k, pl.ds(j * LANES, LANES)]
                        for k in range(K)
                    ]
                    while len(vals) > 1:
                        nxt_vals = [
                            jnp.maximum(vals[t], vals[t + 1])
                            for t in range(0, len(vals) - 1, 2)
                        ]
                        if len(vals) % 2:
                            nxt_vals.append(vals[-1])
                        vals = nxt_vals
                    out_v[i, pl.ds(j * LANES, LANES)] = vals[0]
            pltpu.sync_copy(out_v, out_hbm.at[pl.ds(base + g * CPTS, CPTS)])
            nxt = g + 2

            @pl.when(nxt < NCHUNK)
            def _():
                start(nxt, b)


def _make_pool():
    return pl.kernel(
        _pool_body,
        mesh=plsc.VectorSubcoreMesh(core_axis_name="c", subcore_axis_name="s"),
        out_type=jax.ShapeDtypeStruct((P_PAD, F), jnp.float32),
        scratch_types=[
            pltpu.VMEM((NCHUNK, 128), jnp.int32),
            pltpu.VMEM((2, ROWS, F), jnp.float32),
            pltpu.VMEM((CPTS, F), jnp.float32),
            pltpu.SemaphoreType.DMA,
            pltpu.SemaphoreType.DMA,
        ],
    )


_pool_kernel = _make_pool()


def kernel(points, features, neighbor_indices):
    del points
    idx = neighbor_indices.astype(jnp.int32)
    idx = jnp.pad(idx, ((0, P_PAD - P), (0, 0)))
    idx3 = idx.reshape(NW, NCHUNK, 128)
    out = _pool_kernel(features, idx3)
    return out[:P]


# speedup vs baseline: 1.4670x; 1.0297x over previous
import jax
import jax.numpy as jnp
from jax import lax
from jax.experimental import pallas as pl
from jax.experimental.pallas import tpu as pltpu
from jax.experimental.pallas import tpu_sc as plsc

N = 50000
F = 128
P = 25000
K = 16

NC = 2
NS = 16
NW = NC * NS

P_PAD = 25600
PPW = P_PAD // NW
CPTS = 8
NCHUNK = PPW // CPTS
ROWS = CPTS * K
LANES = 16
NBUF = 4
NOBUF = 2


def _pool_body(features_hbm, idx_hbm, out_hbm, idx_v, rows_v, out_v,
               gs0, gs1, gs2, gs3, os0, os1):
    wid = lax.axis_index("s") * NC + lax.axis_index("c")
    base = wid * PPW
    gsems = (gs0, gs1, gs2, gs3)
    osems = (os0, os1)

    pltpu.sync_copy(idx_hbm.at[wid], idx_v)

    def gather_start(g, b):
        pltpu.make_async_copy(
            features_hbm.at[idx_v.at[g]], rows_v.at[b], gsems[b]
        ).start()

    def gather_wait(b):
        pltpu.make_async_copy(
            features_hbm.at[idx_v.at[0]], rows_v.at[b], gsems[b]
        ).wait()

    def store_start(g, ob):
        pltpu.make_async_copy(
            out_v.at[ob], out_hbm.at[pl.ds(base + g * CPTS, CPTS)], osems[ob]
        ).start()

    def store_wait(ob):
        pltpu.make_async_copy(
            out_v.at[ob], out_hbm.at[pl.ds(base, CPTS)], osems[ob]
        ).wait()

    for b in range(NBUF):
        gather_start(b, b)

    @pl.loop(0, NCHUNK, step=NBUF)
    def _chunks(g4):
        for b in range(NBUF):
            gg = g4 + b
            ob = b % NOBUF
            gather_wait(b)

            @pl.when(gg >= NOBUF)
            def _():
                store_wait(ob)

            @pl.loop(0, CPTS)
            def _pts(i):
                r0 = i * K
                for j in range(F // LANES):
                    col = pl.ds(j * LANES, LANES)
                    acc = None
                    for k0 in range(0, K, 4):
                        v0 = rows_v[b, r0 + k0, col]
                        v1 = rows_v[b, r0 + k0 + 1, col]
                        v2 = rows_v[b, r0 + k0 + 2, col]
                        v3 = rows_v[b, r0 + k0 + 3, col]
                        m = jnp.maximum(jnp.maximum(v0, v1),
                                        jnp.maximum(v2, v3))
                        acc = m if acc is None else jnp.maximum(acc, m)
                    out_v[ob, i, col] = acc

            store_start(gg, ob)
            nxt = gg + NBUF

            @pl.when(nxt < NCHUNK)
            def _():
                gather_start(nxt, b)

    store_wait(0)
    store_wait(1)


_pool_kernel = pl.kernel(
    _pool_body,
    mesh=plsc.VectorSubcoreMesh(core_axis_name="c", subcore_axis_name="s"),
    out_type=jax.ShapeDtypeStruct((P_PAD, F), jnp.float32),
    scratch_types=[
        pltpu.VMEM((NCHUNK, 128), jnp.int32),
        pltpu.VMEM((NBUF, ROWS, F), jnp.float32),
        pltpu.VMEM((NOBUF, CPTS, F), jnp.float32),
        pltpu.SemaphoreType.DMA,
        pltpu.SemaphoreType.DMA,
        pltpu.SemaphoreType.DMA,
        pltpu.SemaphoreType.DMA,
        pltpu.SemaphoreType.DMA,
        pltpu.SemaphoreType.DMA,
    ],
)


def kernel(points, features, neighbor_indices):
    del points
    idx = neighbor_indices.astype(jnp.int32)
    idx = jnp.pad(idx, ((0, P_PAD - P), (0, 0)))
    idx3 = idx.reshape(NW, NCHUNK, 128)
    out = _pool_kernel(features, idx3)
    return out[:P]
